# baseline (speedup 1.0000x reference)
import functools

import jax
import jax.numpy as jnp
from jax import lax
from jax.experimental import pallas as pl
from jax.experimental.pallas import tpu as pltpu
from jax.experimental.pallas import tpu_sc as plsc

N = 10000
NPAD = 10240
NFEAT = 128
NHID = 128
NCLASS = 40
DPAD2 = 64

NC = 2
NS = 16
NW = NC * NS
CH = 128
ROWS_PT = NPAD // NS

_MESH = plsc.VectorSubcoreMesh(
    core_axis_name="c", subcore_axis_name="s", num_cores=NC, num_subcores=NS)


def _rsqrt_newton(x):
  i = plsc.bitcast(x, jnp.int32)
  y = plsc.bitcast(jnp.int32(0x5F3759DF) - (i >> 1), jnp.float32)
  hx = 0.5 * x
  y = y * (1.5 - hx * y * y)
  y = y * (1.5 - hx * y * y)
  y = y * (1.5 - hx * y * y)
  return y


_DEG_W = 2560
_DEG_R = _DEG_W // NS


def _make_deg(cpt):
  def body(col_hbm, ew_hbm, out_hbm, col_v, ew_v, deg_v, acc_v, tmp_v, spbuf):
    cid = lax.axis_index("c")
    sid = lax.axis_index("s")
    wid = sid * NC + cid
    pltpu.sync_copy(col_hbm.at[wid], col_v)
    pltpu.sync_copy(ew_hbm.at[wid], ew_v)

    z16 = jnp.zeros((16,), jnp.float32)

    def zbody(i, _):
      deg_v[pl.ds(i * 16, 16)] = z16
      return 0
    lax.fori_loop(0, NPAD // 16, zbody, 0)

    def sbody(c, _):
      for k in range(CH // 16):
        s = pl.ds(k * 16, 16)
        plsc.addupdate_scatter(deg_v, [col_v[c, s]], ew_v[c, s])
      return 0
    lax.fori_loop(0, cpt, sbody, 0)

    for r in range(NPAD // _DEG_W):
      pltpu.sync_copy(deg_v.at[pl.ds(r * _DEG_W, _DEG_W)], spbuf.at[sid])
      plsc.subcore_barrier()
      pltpu.sync_copy(spbuf.at[0, pl.ds(sid * _DEG_R, _DEG_R)], acc_v)

      def rbody(k, _):
        pltpu.sync_copy(spbuf.at[k, pl.ds(sid * _DEG_R, _DEG_R)], tmp_v)

        def abody(i, _):
          s = pl.ds(i * 16, 16)
          acc_v[s] = acc_v[s] + tmp_v[s]
          return 0
        lax.fori_loop(0, _DEG_R // 16, abody, 0)
        return 0
      lax.fori_loop(1, NS, rbody, 0)

      pltpu.sync_copy(acc_v,
                      out_hbm.at[cid, pl.ds(r * _DEG_W + sid * _DEG_R, _DEG_R)])
      plsc.subcore_barrier()

  return pl.kernel(
      body,
      out_type=jax.ShapeDtypeStruct((NC, NPAD), jnp.float32),
      mesh=_MESH,
      compiler_params=pltpu.CompilerParams(needs_layout_passes=False, use_tc_tiling_on_sc=False),
      scratch_types=[
          pltpu.VMEM((cpt, CH), jnp.int32),
          pltpu.VMEM((cpt, CH), jnp.float32),
          pltpu.VMEM((NPAD,), jnp.float32),
          pltpu.VMEM((_DEG_R,), jnp.float32),
          pltpu.VMEM((_DEG_R,), jnp.float32),
          pltpu.VMEM_SHARED((NS, _DEG_W), jnp.float32),
      ],
  )


DW = 64


def _make_spmm(cpt, nh):
  def body(h_hbm, row_hbm, col_hbm, ew_hbm, deg_hbm, out_hbm,
           row_v, col_v, idx2_v, ew_v, dis_v, dtmp_v,
           rows0, rows1, rows2, accum, sg0, sg1, sg2):
    rbufs = (rows0, rows1, rows2)
    gsems = (sg0, sg1, sg2)
    cid = lax.axis_index("c")
    sid = lax.axis_index("s")
    wid = sid * NC + cid
    pltpu.sync_copy(row_hbm.at[wid], row_v)
    pltpu.sync_copy(col_hbm.at[wid], col_v)
    pltpu.sync_copy(ew_hbm.at[wid], ew_v)
    pltpu.sync_copy(deg_hbm.at[0], dis_v)
    pltpu.sync_copy(deg_hbm.at[1], dtmp_v)

    def disbody(i, _):
      s = pl.ds(i * 16, 16)
      dis_v[s] = _rsqrt_newton(dis_v[s] + dtmp_v[s] + 1.0)
      return 0
    lax.fori_loop(0, NPAD // 16, disbody, 0)

    def nbody(c, _):
      for k in range(CH // 16):
        s = pl.ds(k * 16, 16)
        r = plsc.load_gather(dis_v, [row_v[c, s]])
        cc = plsc.load_gather(dis_v, [col_v[c, s]])
        ew_v[c, s] = r * ew_v[c, s] * cc
      return 0
    lax.fori_loop(0, cpt, nbody, 0)

    z16 = jnp.zeros((16,), jnp.float32)
    base = sid * ROWS_PT

    for fh in range(nh):
      if nh > 1:
        def ibody(c, _):
          for k in range(CH // 16):
            s = pl.ds(k * 16, 16)
            idx2_v[c, s] = row_v[c, s] * nh + fh
          return 0
        lax.fori_loop(0, cpt, ibody, 0)
        idx_ref = idx2_v
      else:
        idx_ref = row_v

      def zbody(i, _):
        for q in range(DW // 16):
          rows0[i, pl.ds(q * 16, 16)] = z16
        return 0
      lax.fori_loop(0, CH, zbody, 0)
      for b in range(ROWS_PT // CH):
        pltpu.sync_copy(rows0, accum.at[pl.ds(base + b * CH, CH)])
      plsc.subcore_barrier()

      pltpu.async_copy(h_hbm.at[idx_ref.at[0]], rows0, sg0)
      pltpu.async_copy(h_hbm.at[idx_ref.at[1]], rows1, sg1)

      def gbody(j, _):
        for b in range(3):
          rv, sg = rbufs[b], gsems[b]
          nb = (b + 2) % 3
          rv2, sg2 = rbufs[nb], gsems[nb]
          c = j * 3 + b

          @pl.when(c + 2 < cpt)
          def _():
            pltpu.async_copy(h_hbm.at[idx_ref.at[c + 2]], rv2, sg2)

          pltpu.make_async_copy(h_hbm.at[idx_ref.at[c]], rv, sg).wait()

          def scbody(i, _):
            bi = jnp.zeros((16,), jnp.int32) + i
            nv = plsc.load_gather(ew_v.at[c], [bi])
            for q in range(DW // 16):
              s = pl.ds(q * 16, 16)
              rv[i, s] = rv[i, s] * nv
            return 0
          lax.fori_loop(0, CH, scbody, 0)

          pltpu.sync_copy(rv, accum.at[col_v.at[c]], add=True)
        return 0
      lax.fori_loop(0, cpt // 3, gbody, 0)

      plsc.subcore_barrier()
      pltpu.sync_copy(accum.at[pl.ds(base, ROWS_PT)],
                      out_hbm.at[cid, fh, pl.ds(base, ROWS_PT)])
      if fh + 1 < nh:
        plsc.subcore_barrier()

  return pl.kernel(
      body,
      out_type=jax.ShapeDtypeStruct((NC, nh, NPAD, DW), jnp.float32),
      mesh=_MESH,
      compiler_params=pltpu.CompilerParams(needs_layout_passes=False, use_tc_tiling_on_sc=False),
      scratch_types=[
          pltpu.VMEM((cpt, CH), jnp.int32),
          pltpu.VMEM((cpt, CH), jnp.int32),
          pltpu.VMEM((cpt, CH), jnp.int32),
          pltpu.VMEM((cpt, CH), jnp.float32),
          pltpu.VMEM((NPAD,), jnp.float32),
          pltpu.VMEM((NPAD,), jnp.float32),
          pltpu.VMEM((CH, DW), jnp.float32),
          pltpu.VMEM((CH, DW), jnp.float32),
          pltpu.VMEM((CH, DW), jnp.float32),
          pltpu.VMEM_SHARED((NPAD, DW), jnp.float32),
          pltpu.SemaphoreType.DMA,
          pltpu.SemaphoreType.DMA,
          pltpu.SemaphoreType.DMA,
      ],
  )


_R = 1280


def _mm1(xp, W1):
  def body(x_ref, w_ref, o_ref):
    o_ref[...] = jnp.dot(x_ref[...], w_ref[...],
                         preferred_element_type=jnp.float32)
  return pl.pallas_call(
      body,
      out_shape=jax.ShapeDtypeStruct((NPAD, NHID), jnp.float32),
      grid=(NPAD // _R,),
      in_specs=[pl.BlockSpec((_R, NFEAT), lambda i: (i, 0)),
                pl.BlockSpec((NFEAT, NHID), lambda i: (0, 0))],
      out_specs=pl.BlockSpec((_R, NHID), lambda i: (i, 0)),
  )(xp, W1)


def _mid(agg, h, degp, b1, W2p):
  def body(a_ref, h_ref, d_ref, b_ref, w_ref, o_ref):
    inv = 1.0 / (d_ref[0] + d_ref[1] + 1.0)
    t0 = (a_ref[0, 0] + a_ref[1, 0] + inv[:, None] * h_ref[:, :DW]
          + b_ref[...][None, :DW])
    t1 = (a_ref[0, 1] + a_ref[1, 1] + inv[:, None] * h_ref[:, DW:]
          + b_ref[...][None, DW:])
    h1a = jnp.maximum(t0, 0.0)
    h1b = jnp.maximum(t1, 0.0)
    o_ref[...] = (
        jnp.dot(h1a, w_ref[:DW], preferred_element_type=jnp.float32)
        + jnp.dot(h1b, w_ref[DW:], preferred_element_type=jnp.float32))
  return pl.pallas_call(
      body,
      out_shape=jax.ShapeDtypeStruct((NPAD, DPAD2), jnp.float32),
      grid=(NPAD // _R,),
      in_specs=[pl.BlockSpec((NC, 2, _R, DW), lambda i: (0, 0, i, 0)),
                pl.BlockSpec((_R, NHID), lambda i: (i, 0)),
                pl.BlockSpec((NC, _R), lambda i: (0, i)),
                pl.BlockSpec((NHID,), lambda i: (0,)),
                pl.BlockSpec((NHID, DPAD2), lambda i: (0, 0))],
      out_specs=pl.BlockSpec((_R, DPAD2), lambda i: (i, 0)),
  )(agg, h, degp, b1, W2p)


def _fin(agg, g, degp, b2p):
  def body(a_ref, g_ref, d_ref, b_ref, o_ref):
    inv = 1.0 / (d_ref[0] + d_ref[1] + 1.0)
    z = (a_ref[0, 0] + a_ref[1, 0] + inv[:, None] * g_ref[...]
         + b_ref[...][None, :])
    mask = lax.broadcasted_iota(jnp.int32, (_R, DPAD2), 1) < NCLASS
    zneg = jnp.where(mask, z, -jnp.inf)
    m = jnp.max(zneg, axis=1, keepdims=True)
    e = jnp.where(mask, jnp.exp(z - m), 0.0)
    lse = jnp.log(jnp.sum(e, axis=1, keepdims=True)) + m
    o_ref[...] = z - lse
  return pl.pallas_call(
      body,
      out_shape=jax.ShapeDtypeStruct((NPAD, DPAD2), jnp.float32),
      grid=(NPAD // _R,),
      in_specs=[pl.BlockSpec((NC, 1, _R, DPAD2), lambda i: (0, 0, i, 0)),
                pl.BlockSpec((_R, DPAD2), lambda i: (i, 0)),
                pl.BlockSpec((NC, _R), lambda i: (0, i)),
                pl.BlockSpec((DPAD2,), lambda i: (0,))],
      out_specs=pl.BlockSpec((_R, DPAD2), lambda i: (i, 0)),
  )(agg, g, degp, b2p)


def kernel(x, edge_index, edge_weight, W1, b1, W2, b2):
  e = edge_index.shape[1]
  cpt = -(-e // (NW * CH))
  cpt = -(-cpt // 3) * 3
  epad = NW * cpt * CH
  pad = epad - e

  row = jnp.concatenate([edge_index[0], jnp.zeros((pad,), jnp.int32)])
  col = jnp.concatenate([edge_index[1], jnp.zeros((pad,), jnp.int32)])
  ew = jnp.concatenate([edge_weight, jnp.zeros((pad,), jnp.float32)])
  rowp = row.reshape(NW, cpt, CH)
  colp = col.reshape(NW, cpt, CH)
  ewp = ew.reshape(NW, cpt, CH)
  xp = jnp.pad(x, ((0, NPAD - N), (0, 0)))
  W2p = jnp.pad(W2, ((0, 0), (0, DPAD2 - NCLASS)))
  b2p = jnp.pad(b2, (0, DPAD2 - NCLASS))

  degp = _make_deg(cpt)(colp, ewp)
  h = _mm1(xp, W1)
  h2 = h.reshape(2 * NPAD, DW)
  agg1 = _make_spmm(cpt, 2)(h2, rowp, colp, ewp, degp)
  g = _mid(agg1, h, degp, b1, W2p)
  agg2 = _make_spmm(cpt, 1)(g, rowp, colp, ewp, degp)
  fin = _fin(agg2, g, degp, b2p)
  return fin[:N, :NCLASS]


# speedup vs baseline: 1.5253x; 1.5253x over previous
import jax
import jax.numpy as jnp
from jax import lax
from jax.experimental import pallas as pl
from jax.experimental.pallas import tpu as pltpu
from jax.experimental.pallas import tpu_sc as plsc

N = 10000
NPAD = 10240
NFEAT = 128
NHID = 128
NCLASS = 40
DW = 64
DPAD2 = 64

NC = 2
NS = 16
NW = NC * NS
CH = 128
ROWS_PT = NPAD // NS

_MESH = plsc.VectorSubcoreMesh(
    core_axis_name="c", subcore_axis_name="s", num_cores=NC, num_subcores=NS)

_SC_PARAMS = pltpu.CompilerParams(needs_layout_passes=False,
                                  use_tc_tiling_on_sc=False)


def _rsqrt_newton(x):
  i = plsc.bitcast(x, jnp.int32)
  y = plsc.bitcast(jnp.int32(0x5F3759DF) - (i >> 1), jnp.float32)
  hx = 0.5 * x
  y = y * (1.5 - hx * y * y)
  y = y * (1.5 - hx * y * y)
  y = y * (1.5 - hx * y * y)
  return y


def _interleave64(t):
  n = t.shape[0]
  return t.reshape(n, 2, 2, 16).transpose(0, 1, 3, 2).reshape(n, 64)


_DEG_W = 1280
_DEG_R = _DEG_W // NS


def _make_deg(cpt):
  def body(col_hbm, ew_hbm, out_hbm, col_v, ew_v, deg_v, acc_v, tmp_v, spbuf):
    cid = lax.axis_index("c")
    sid = lax.axis_index("s")
    wid = sid * NC + cid
    pltpu.sync_copy(col_hbm.at[wid], col_v)
    pltpu.sync_copy(ew_hbm.at[wid], ew_v)

    z16 = jnp.zeros((16,), jnp.float32)

    def zbody(i, _):
      deg_v[pl.ds(i * 16, 16)] = z16
      return 0
    lax.fori_loop(0, NPAD // 16, zbody, 0)

    def sbody(c, _):
      for k in range(CH // 16):
        s = pl.ds(k * 16, 16)
        plsc.addupdate_scatter(deg_v, [col_v[c, s]], ew_v[c, s])
      return 0
    lax.fori_loop(0, cpt, sbody, 0)

    for r in range(NPAD // _DEG_W):
      pltpu.sync_copy(deg_v.at[pl.ds(r * _DEG_W, _DEG_W)], spbuf.at[sid])
      plsc.subcore_barrier()
      pltpu.sync_copy(spbuf.at[0, pl.ds(sid * _DEG_R, _DEG_R)], acc_v)

      def rbody(k, _):
        pltpu.sync_copy(spbuf.at[k, pl.ds(sid * _DEG_R, _DEG_R)], tmp_v)

        def abody(i, _):
          s = pl.ds(i * 16, 16)
          acc_v[s] = acc_v[s] + tmp_v[s]
          return 0
        lax.fori_loop(0, _DEG_R // 16, abody, 0)
        return 0
      lax.fori_loop(1, NS, rbody, 0)

      pltpu.sync_copy(acc_v,
                      out_hbm.at[cid, pl.ds(r * _DEG_W + sid * _DEG_R, _DEG_R)])
      plsc.subcore_barrier()

  return pl.kernel(
      body,
      out_type=jax.ShapeDtypeStruct((NC, NPAD), jnp.float32),
      mesh=_MESH,
      compiler_params=_SC_PARAMS,
      scratch_types=[
          pltpu.VMEM((cpt, CH), jnp.int32),
          pltpu.VMEM((cpt, CH), jnp.float32),
          pltpu.VMEM((NPAD,), jnp.float32),
          pltpu.VMEM((_DEG_R,), jnp.float32),
          pltpu.VMEM((_DEG_R,), jnp.float32),
          pltpu.VMEM_SHARED((NS, _DEG_W), jnp.float32),
      ],
  )


def _make_spmm(cpt, nh):
  def body(h_hbm, row_hbm, col_hbm, ew_hbm, deg_hbm, out_hbm,
           row_v, col_v, idx2_v, ew_v, dis_v, dtmp_v,
           rvb0, rvb1, stage, accum, sg0, sg1):
    rbufs = (rvb0, rvb1)
    gsems = (sg0, sg1)
    cid = lax.axis_index("c")
    sid = lax.axis_index("s")
    wid = sid * NC + cid
    pltpu.sync_copy(row_hbm.at[wid], row_v)
    pltpu.sync_copy(col_hbm.at[wid], col_v)
    pltpu.sync_copy(ew_hbm.at[wid], ew_v)
    pltpu.sync_copy(deg_hbm.at[0], dis_v)
    pltpu.sync_copy(deg_hbm.at[1], dtmp_v)

    def disbody(i, _):
      s = pl.ds(i * 16, 16)
      dis_v[s] = _rsqrt_newton(dis_v[s] + dtmp_v[s] + 1.0)
      return 0
    lax.fori_loop(0, NPAD // 16, disbody, 0)

    def nbody(c, _):
      for k in range(CH // 16):
        s = pl.ds(k * 16, 16)
        r = plsc.load_gather(dis_v, [row_v[c, s]])
        cc = plsc.load_gather(dis_v, [col_v[c, s]])
        ew_v[c, s] = r * ew_v[c, s] * cc
      return 0
    lax.fori_loop(0, cpt, nbody, 0)

    z16 = jnp.zeros((16,), jnp.float32)
    base = sid * ROWS_PT

    for fh in range(nh):
      if nh > 1:
        def ibody(c, _):
          for k in range(CH // 16):
            s = pl.ds(k * 16, 16)
            idx2_v[c, s] = row_v[c, s] * nh + fh
          return 0
        lax.fori_loop(0, cpt, ibody, 0)
        idx_ref = idx2_v
      else:
        idx_ref = row_v

      def zbody(i, _):
        for q in range(DW // 16):
          stage[i, pl.ds(q * 16, 16)] = z16
        return 0
      lax.fori_loop(0, CH, zbody, 0)
      for b in range(ROWS_PT // CH):
        pltpu.sync_copy(stage, accum.at[pl.ds(base + b * CH, CH)])
      plsc.subcore_barrier()

      pltpu.async_copy(h_hbm.at[idx_ref.at[0]], rvb0, sg0)

      def gbody(j, _):
        for b in range(2):
          rv, sg = rbufs[b], gsems[b]
          ov, og = rbufs[1 - b], gsems[1 - b]
          c = j * 2 + b

          @pl.when(c + 1 < cpt)
          def _():
            pltpu.async_copy(h_hbm.at[idx_ref.at[c + 1]], ov, og)

          pltpu.make_async_copy(h_hbm.at[idx_ref.at[c]], rv, sg).wait()

          def scbody(i, _):
            bi = jnp.zeros((16,), jnp.int32) + i
            nv = plsc.load_gather(ew_v.at[c], [bi])
            for g in range(DW // 32):
              w = rv[i, pl.ds(g * 32, 32)]
              a, bb = plsc.unpack(w, format=plsc.PackFormat.INTERLEAVED)
              stage[i, pl.ds(g * 32, 16)] = a * nv
              stage[i, pl.ds(g * 32 + 16, 16)] = bb * nv
            return 0
          lax.fori_loop(0, CH, scbody, 0)

          pltpu.sync_copy(stage, accum.at[col_v.at[c]], add=True)
        return 0
      lax.fori_loop(0, cpt // 2, gbody, 0)

      plsc.subcore_barrier()
      pltpu.sync_copy(accum.at[pl.ds(base, ROWS_PT)],
                      out_hbm.at[cid, fh, pl.ds(base, ROWS_PT)])
      if fh + 1 < nh:
        plsc.subcore_barrier()

  return pl.kernel(
      body,
      out_type=jax.ShapeDtypeStruct((NC, nh, NPAD, DW), jnp.float32),
      mesh=_MESH,
      compiler_params=_SC_PARAMS,
      scratch_types=[
          pltpu.VMEM((cpt, CH), jnp.int32),
          pltpu.VMEM((cpt, CH), jnp.int32),
          pltpu.VMEM((cpt, CH), jnp.int32),
          pltpu.VMEM((cpt, CH), jnp.float32),
          pltpu.VMEM((NPAD,), jnp.float32),
          pltpu.VMEM((NPAD,), jnp.float32),
          pltpu.VMEM((CH, DW), jnp.bfloat16),
          pltpu.VMEM((CH, DW), jnp.bfloat16),
          pltpu.VMEM((CH, DW), jnp.float32),
          pltpu.VMEM_SHARED((NPAD, DW), jnp.float32),
          pltpu.SemaphoreType.DMA,
          pltpu.SemaphoreType.DMA,
      ],
  )


_R = 1280


def _mm1(xp, W1):
  def body(x_ref, w_ref, o_ref):
    o_ref[...] = jnp.dot(x_ref[...], w_ref[...],
                         preferred_element_type=jnp.float32)
  return pl.pallas_call(
      body,
      out_shape=jax.ShapeDtypeStruct((NPAD, NHID), jnp.float32),
      grid=(NPAD // _R,),
      in_specs=[pl.BlockSpec((_R, NFEAT), lambda i: (i, 0)),
                pl.BlockSpec((NFEAT, NHID), lambda i: (0, 0))],
      out_specs=pl.BlockSpec((_R, NHID), lambda i: (i, 0)),
  )(xp, W1)


def _mid(agg, h, degp, b1, W2p):
  def body(a_ref, h_ref, d_ref, b_ref, w_ref, o_ref):
    inv = 1.0 / (d_ref[0] + d_ref[1] + 1.0)
    t0 = (a_ref[0, 0] + a_ref[1, 0] + inv[:, None] * h_ref[:, :DW]
          + b_ref[...][None, :DW])
    t1 = (a_ref[0, 1] + a_ref[1, 1] + inv[:, None] * h_ref[:, DW:]
          + b_ref[...][None, DW:])
    h1a = jnp.maximum(t0, 0.0)
    h1b = jnp.maximum(t1, 0.0)
    o_ref[...] = (
        jnp.dot(h1a, w_ref[:DW], preferred_element_type=jnp.float32)
        + jnp.dot(h1b, w_ref[DW:], preferred_element_type=jnp.float32))
  return pl.pallas_call(
      body,
      out_shape=jax.ShapeDtypeStruct((NPAD, DPAD2), jnp.float32),
      grid=(NPAD // _R,),
      in_specs=[pl.BlockSpec((NC, 2, _R, DW), lambda i: (0, 0, i, 0)),
                pl.BlockSpec((_R, NHID), lambda i: (i, 0)),
                pl.BlockSpec((NC, _R), lambda i: (0, i)),
                pl.BlockSpec((NHID,), lambda i: (0,)),
                pl.BlockSpec((NHID, DPAD2), lambda i: (0, 0))],
      out_specs=pl.BlockSpec((_R, DPAD2), lambda i: (i, 0)),
  )(agg, h, degp, b1, W2p)


def _fin(agg, g, degp, b2p):
  def body(a_ref, g_ref, d_ref, b_ref, o_ref):
    inv = 1.0 / (d_ref[0] + d_ref[1] + 1.0)
    z = (a_ref[0, 0] + a_ref[1, 0] + inv[:, None] * g_ref[...]
         + b_ref[...][None, :])
    mask = lax.broadcasted_iota(jnp.int32, (_R, DPAD2), 1) < NCLASS
    zneg = jnp.where(mask, z, -jnp.inf)
    m = jnp.max(zneg, axis=1, keepdims=True)
    e = jnp.where(mask, jnp.exp(z - m), 0.0)
    lse = jnp.log(jnp.sum(e, axis=1, keepdims=True)) + m
    o_ref[...] = z - lse
  return pl.pallas_call(
      body,
      out_shape=jax.ShapeDtypeStruct((NPAD, DPAD2), jnp.float32),
      grid=(NPAD // _R,),
      in_specs=[pl.BlockSpec((NC, 1, _R, DPAD2), lambda i: (0, 0, i, 0)),
                pl.BlockSpec((_R, DPAD2), lambda i: (i, 0)),
                pl.BlockSpec((NC, _R), lambda i: (0, i)),
                pl.BlockSpec((DPAD2,), lambda i: (0,))],
      out_specs=pl.BlockSpec((_R, DPAD2), lambda i: (i, 0)),
  )(agg, g, degp, b2p)


def kernel(x, edge_index, edge_weight, W1, b1, W2, b2):
  e = edge_index.shape[1]
  cpt = -(-e // (NW * CH))
  cpt = -(-cpt // 2) * 2
  epad = NW * cpt * CH
  pad = epad - e

  row = jnp.concatenate([edge_index[0], jnp.zeros((pad,), jnp.int32)])
  col = jnp.concatenate([edge_index[1], jnp.zeros((pad,), jnp.int32)])
  ew = jnp.concatenate([edge_weight, jnp.zeros((pad,), jnp.float32)])
  rowp = row.reshape(NW, cpt, CH)
  colp = col.reshape(NW, cpt, CH)
  ewp = ew.reshape(NW, cpt, CH)
  xp = jnp.pad(x, ((0, NPAD - N), (0, 0)))
  W2p = jnp.pad(W2, ((0, 0), (0, DPAD2 - NCLASS)))
  b2p = jnp.pad(b2, (0, DPAD2 - NCLASS))

  degp = _make_deg(cpt)(colp, ewp)
  h = _mm1(xp, W1)
  hb = _interleave64(h.reshape(2 * NPAD, DW)).astype(jnp.bfloat16)
  agg1 = _make_spmm(cpt, 2)(hb, rowp, colp, ewp, degp)
  g = _mid(agg1, h, degp, b1, W2p)
  gb = _interleave64(g).astype(jnp.bfloat16)
  agg2 = _make_spmm(cpt, 1)(gb, rowp, colp, ewp, degp)
  fin = _fin(agg2, g, degp, b2p)
  return fin[:N, :NCLASS]
